# Initial kernel scaffold; baseline (speedup 1.0000x reference)
import jax
import jax.numpy as jnp
from jax.experimental import pallas as pl


def kernel(hidden_states, W_router, Wg, Wu, Wd):
    raise NotImplementedError("write your pallas kernel here")


# speedup vs baseline: 1.1110x; 1.1110x over previous
import functools

import jax
import jax.numpy as jnp
from jax import lax
from jax.experimental import pallas as pl
from jax.experimental.pallas import tpu as pltpu
from jax.experimental.pallas import tpu_sc as plsc

D = 2048
F = 4096
E = 8
K = 2
T = 4096
BLK = 512
NB = 24
P = NB * BLK
FT = 512
NF = F // FT

NC = 2
NS = 16
NW = NC * NS
L = 16

RPW = P // NW
TPW = T // NW


def _router_body(x_ref, wr_ref, logits_ref, wt_ref, idx_ref):
    xb = x_ref[...]
    wr = wr_ref[...]
    logits = lax.dot_general(
        xb.astype(jnp.bfloat16), wr.astype(jnp.bfloat16),
        (((1,), (1,)), ((), ())),
        preferred_element_type=jnp.float32,
    )
    m = jnp.max(logits, axis=1, keepdims=True)
    p = jnp.exp(logits - m)
    probs = p / jnp.sum(p, axis=1, keepdims=True)
    eio = lax.broadcasted_iota(jnp.int32, probs.shape, 1)
    w1 = jnp.max(probs, axis=1, keepdims=True)
    i1 = jnp.min(jnp.where(probs == w1, eio, E), axis=1, keepdims=True)
    probs2 = jnp.where(eio == i1, -1.0, probs)
    w2 = jnp.max(probs2, axis=1, keepdims=True)
    i2 = jnp.min(jnp.where(probs2 == w2, eio, E), axis=1, keepdims=True)
    logits_ref[...] = logits
    wt_ref[...] = jnp.concatenate([w1, w2], axis=1)
    idx_ref[...] = jnp.concatenate([i1, i2], axis=1)


def _router(x, wr):
    RB = 1024
    return pl.pallas_call(
        _router_body,
        grid=(T // RB,),
        in_specs=[
            pl.BlockSpec((RB, D), lambda r: (r, 0)),
            pl.BlockSpec((E, D), lambda r: (0, 0)),
        ],
        out_specs=[
            pl.BlockSpec((RB, E), lambda r: (r, 0)),
            pl.BlockSpec((RB, K), lambda r: (r, 0)),
            pl.BlockSpec((RB, K), lambda r: (r, 0)),
        ],
        out_shape=[
            jax.ShapeDtypeStruct((T, E), jnp.float32),
            jax.ShapeDtypeStruct((T, K), jnp.float32),
            jax.ShapeDtypeStruct((T, K), jnp.int32),
        ],
    )(x, wr)


def _plan_body(idx_ref, pos_ref, be_ref, bv_ref):
    i1 = idx_ref[:, 0:1]
    i2 = idx_ref[:, 1:2]
    eio = lax.broadcasted_iota(jnp.int32, (T, E), 1)
    m = ((i1 == eio) | (i2 == eio)).astype(jnp.int32)
    cums = m
    sh = 1
    while sh < T:
        shifted = jnp.concatenate(
            [jnp.zeros((sh, E), jnp.int32), cums[: T - sh, :]], axis=0)
        cums = cums + shifted
        sh *= 2
    counts = cums[T - 1:T, :]
    padded = ((counts + BLK - 1) // BLK) * BLK
    tri = (lax.broadcasted_iota(jnp.int32, (E, E), 0)
           < lax.broadcasted_iota(jnp.int32, (E, E), 1)).astype(jnp.float32)
    starts = lax.dot_general(
        padded.astype(jnp.float32), tri, (((1,), (0,)), ((), ())),
        preferred_element_type=jnp.float32).astype(jnp.int32)
    startsb = jnp.broadcast_to(starts, (T, E))
    c1 = jnp.sum(jnp.where(eio == i1, cums, 0), axis=1, keepdims=True)
    s1 = jnp.sum(jnp.where(eio == i1, startsb, 0), axis=1, keepdims=True)
    c2 = jnp.sum(jnp.where(eio == i2, cums, 0), axis=1, keepdims=True)
    s2 = jnp.sum(jnp.where(eio == i2, startsb, 0), axis=1, keepdims=True)
    pos_ref[...] = jnp.concatenate([s1 + c1 - 1, s2 + c2 - 1], axis=1)
    sb = lax.broadcasted_iota(jnp.int32, (NB, 1), 0) * BLK
    startsnb = jnp.broadcast_to(starts, (NB, E))
    be_ref[...] = jnp.sum((startsnb <= sb).astype(jnp.int32),
                          axis=1, keepdims=True) - 1
    total = jnp.sum(padded, axis=1, keepdims=True)
    bv_ref[...] = (sb < total).astype(jnp.int32)


def _plan(idx):
    return pl.pallas_call(
        _plan_body,
        out_shape=[
            jax.ShapeDtypeStruct((T, K), jnp.int32),
            jax.ShapeDtypeStruct((NB, 1), jnp.int32),
            jax.ShapeDtypeStruct((NB, 1), jnp.int32),
        ],
    )(idx)


def _dispatch_body(pos_hbm, xi_hbm, xs_hbm, pos_v, ids_v, row_buf, sem):
    wid = lax.axis_index("s") * NC + lax.axis_index("c")
    pltpu.sync_copy(pos_hbm, pos_v)
    lanes = lax.iota(jnp.int32, L)

    def scatter_i(i, carry):
        pchunk = pos_v[pl.ds(i * L, L)]
        toks = (i * L + lanes) >> 1
        plsc.store_scatter(ids_v, [pchunk], toks)
        return carry

    lax.fori_loop(0, (T * K) // L, scatter_i, 0)

    base = wid * RPW

    def gather_j(j, carry):
        ids16 = ids_v[pl.ds(base + j * L, L)]
        ids16 = jnp.clip(ids16, 0, T - 1)
        pltpu.async_copy(xi_hbm.at[ids16], row_buf, sem).wait()
        pltpu.sync_copy(row_buf, xs_hbm.at[pl.ds(base + j * L, L)])
        return carry

    lax.fori_loop(0, RPW // L, gather_j, 0)


def _dispatch(pos_flat, x_i32):
    mesh = plsc.VectorSubcoreMesh(core_axis_name="c", subcore_axis_name="s")
    return pl.kernel(
        _dispatch_body,
        out_type=jax.ShapeDtypeStruct((P, D // 2), jnp.int32),
        mesh=mesh,
        compiler_params=pltpu.CompilerParams(needs_layout_passes=False),
        scratch_types=[
            pltpu.VMEM((T * K,), jnp.int32),
            pltpu.VMEM((P,), jnp.int32),
            pltpu.VMEM((L, D // 2), jnp.int32),
            pltpu.SemaphoreType.DMA,
        ],
    )(pos_flat, x_i32)


def _ffn_body(be_ref, bv_ref, xs_ref, wg_ref, wu_ref, wd_ref, h_ref):
    f = pl.program_id(1)
    b = pl.program_id(0)

    @pl.when(f == 0)
    def _():
        h_ref[...] = jnp.zeros_like(h_ref)

    @pl.when(bv_ref[b] != 0)
    def _():
        xb = xs_ref[...]
        g = lax.dot_general(xb, wg_ref[0], (((1,), (1,)), ((), ())),
                            preferred_element_type=jnp.float32)
        u = lax.dot_general(xb, wu_ref[0], (((1,), (1,)), ((), ())),
                            preferred_element_type=jnp.float32)
        a = (g * jax.nn.sigmoid(g) * u).astype(jnp.bfloat16)
        h_ref[...] += lax.dot_general(a, wd_ref[0], (((1,), (1,)), ((), ())),
                                      preferred_element_type=jnp.float32)


def _ffn(be, bv, xs, wg, wu, wd):
    grid_spec = pltpu.PrefetchScalarGridSpec(
        num_scalar_prefetch=2,
        grid=(NB, NF),
        in_specs=[
            pl.BlockSpec((BLK, D), lambda b, f, be, bv: (b, 0)),
            pl.BlockSpec((1, FT, D), lambda b, f, be, bv: (be[b], f, 0)),
            pl.BlockSpec((1, FT, D), lambda b, f, be, bv: (be[b], f, 0)),
            pl.BlockSpec((1, D, FT), lambda b, f, be, bv: (be[b], 0, f)),
        ],
        out_specs=pl.BlockSpec((BLK, D), lambda b, f, be, bv: (b, 0)),
    )
    return pl.pallas_call(
        _ffn_body,
        grid_spec=grid_spec,
        out_shape=jax.ShapeDtypeStruct((P, D), jnp.float32),
        compiler_params=pltpu.CompilerParams(
            dimension_semantics=("arbitrary", "arbitrary")),
    )(be, bv, xs, wg, wu, wd)


def _combine_body(pos_hbm, w_hbm, h_hbm, out_hbm,
                  pos_v, w_v, h1_v, h2_v, out_v, sem):
    wid = lax.axis_index("s") * NC + lax.axis_index("c")
    tbase = wid * TPW
    pltpu.sync_copy(pos_hbm.at[pl.ds(tbase * K, TPW * K)], pos_v)
    pltpu.sync_copy(w_hbm.at[pl.ds(tbase * K, TPW * K)], w_v)
    lanes = lax.iota(jnp.int32, L)

    def chunk_j(j, carry):
        pos1 = plsc.load_gather(pos_v, [j * (K * L) + K * lanes])
        pos2 = plsc.load_gather(pos_v, [j * (K * L) + K * lanes + 1])
        pltpu.async_copy(h_hbm.at[pos1], h1_v, sem).wait()
        pltpu.async_copy(h_hbm.at[pos2], h2_v, sem).wait()
        for l in range(L):
            w1 = plsc.load_gather(w_v, [jnp.full((L,), j * (K * L) + K * l,
                                                 jnp.int32)])
            w2 = plsc.load_gather(w_v, [jnp.full((L,), j * (K * L) + K * l + 1,
                                                 jnp.int32)])

            def col_c(c, carry2):
                out_v[l, pl.ds(c * L, L)] = (
                    h1_v[l, pl.ds(c * L, L)] * w1
                    + h2_v[l, pl.ds(c * L, L)] * w2)
                return carry2

            lax.fori_loop(0, D // L, col_c, 0)
        pltpu.sync_copy(out_v, out_hbm.at[pl.ds(tbase + j * L, L)])
        return carry

    lax.fori_loop(0, TPW // L, chunk_j, 0)


def _combine(pos_flat, w_flat, h):
    mesh = plsc.VectorSubcoreMesh(core_axis_name="c", subcore_axis_name="s")
    return pl.kernel(
        _combine_body,
        out_type=jax.ShapeDtypeStruct((T, D), jnp.float32),
        mesh=mesh,
        compiler_params=pltpu.CompilerParams(needs_layout_passes=False,
                                             use_tc_tiling_on_sc=False),
        scratch_types=[
            pltpu.VMEM((TPW * K,), jnp.int32),
            pltpu.VMEM((TPW * K,), jnp.float32),
            pltpu.VMEM((L, D), jnp.float32),
            pltpu.VMEM((L, D), jnp.float32),
            pltpu.VMEM((L, D), jnp.float32),
            pltpu.SemaphoreType.DMA,
        ],
    )(pos_flat, w_flat, h)


@jax.jit
def kernel(hidden_states, W_router, Wg, Wu, Wd):
    bsz, seq, _ = hidden_states.shape
    x = hidden_states.reshape(T, D)
    logits, wt, tidx = _router(x, W_router)
    pos, be, bv = _plan(tidx)
    x_i32 = lax.bitcast_convert_type(
        x.astype(jnp.bfloat16).reshape(T, D // 2, 2), jnp.int32)
    xs_i32 = _dispatch(pos.reshape(-1), x_i32)
    xs = lax.bitcast_convert_type(xs_i32, jnp.bfloat16).reshape(P, D)
    h = _ffn(be.reshape(NB), bv.reshape(NB), xs,
             Wg.astype(jnp.bfloat16), Wu.astype(jnp.bfloat16),
             Wd.astype(jnp.bfloat16))
    out = _combine(pos.reshape(-1), wt.reshape(-1), h)
    return out.reshape(bsz, seq, D), logits


# speedup vs baseline: 1.2598x; 1.1339x over previous
import functools

import jax
import jax.numpy as jnp
from jax import lax
from jax.experimental import pallas as pl
from jax.experimental.pallas import tpu as pltpu
from jax.experimental.pallas import tpu_sc as plsc

D = 2048
F = 4096
E = 8
K = 2
T = 4096
BLK = 512
NB = 24
P = NB * BLK
FT = 512
NF = F // FT

NC = 2
NS = 16
NW = NC * NS
L = 16

RPW = P // NW
TPW = T // NW


def _router_body(x_ref, wr_ref, logits_ref, wt_ref, idx_ref):
    xb = x_ref[...]
    wr = wr_ref[...]
    logits = lax.dot_general(
        xb.astype(jnp.bfloat16), wr.astype(jnp.bfloat16),
        (((1,), (1,)), ((), ())),
        preferred_element_type=jnp.float32,
    )
    m = jnp.max(logits, axis=1, keepdims=True)
    p = jnp.exp(logits - m)
    probs = p / jnp.sum(p, axis=1, keepdims=True)
    eio = lax.broadcasted_iota(jnp.int32, probs.shape, 1)
    w1 = jnp.max(probs, axis=1, keepdims=True)
    i1 = jnp.min(jnp.where(probs == w1, eio, E), axis=1, keepdims=True)
    probs2 = jnp.where(eio == i1, -1.0, probs)
    w2 = jnp.max(probs2, axis=1, keepdims=True)
    i2 = jnp.min(jnp.where(probs2 == w2, eio, E), axis=1, keepdims=True)
    logits_ref[...] = logits
    wt_ref[...] = jnp.concatenate([w1, w2], axis=1)
    idx_ref[...] = jnp.concatenate([i1, i2], axis=1)


def _router(x, wr):
    RB = 1024
    return pl.pallas_call(
        _router_body,
        grid=(T // RB,),
        in_specs=[
            pl.BlockSpec((RB, D), lambda r: (r, 0)),
            pl.BlockSpec((E, D), lambda r: (0, 0)),
        ],
        out_specs=[
            pl.BlockSpec((RB, E), lambda r: (r, 0)),
            pl.BlockSpec((RB, K), lambda r: (r, 0)),
            pl.BlockSpec((RB, K), lambda r: (r, 0)),
        ],
        out_shape=[
            jax.ShapeDtypeStruct((T, E), jnp.float32),
            jax.ShapeDtypeStruct((T, K), jnp.float32),
            jax.ShapeDtypeStruct((T, K), jnp.int32),
        ],
    )(x, wr)


def _plan_body(idx_ref, pos_ref, be_ref, bv_ref):
    i1 = idx_ref[:, 0:1]
    i2 = idx_ref[:, 1:2]
    eio = lax.broadcasted_iota(jnp.int32, (T, E), 1)
    m = ((i1 == eio) | (i2 == eio)).astype(jnp.int32)
    cums = m
    sh = 1
    while sh < T:
        shifted = jnp.concatenate(
            [jnp.zeros((sh, E), jnp.int32), cums[: T - sh, :]], axis=0)
        cums = cums + shifted
        sh *= 2
    counts = cums[T - 1:T, :]
    padded = ((counts + BLK - 1) // BLK) * BLK
    tri = (lax.broadcasted_iota(jnp.int32, (E, E), 0)
           < lax.broadcasted_iota(jnp.int32, (E, E), 1)).astype(jnp.float32)
    starts = lax.dot_general(
        padded.astype(jnp.float32), tri, (((1,), (0,)), ((), ())),
        preferred_element_type=jnp.float32).astype(jnp.int32)
    startsb = jnp.broadcast_to(starts, (T, E))
    c1 = jnp.sum(jnp.where(eio == i1, cums, 0), axis=1, keepdims=True)
    s1 = jnp.sum(jnp.where(eio == i1, startsb, 0), axis=1, keepdims=True)
    c2 = jnp.sum(jnp.where(eio == i2, cums, 0), axis=1, keepdims=True)
    s2 = jnp.sum(jnp.where(eio == i2, startsb, 0), axis=1, keepdims=True)
    pos_ref[...] = jnp.concatenate([s1 + c1 - 1, s2 + c2 - 1], axis=1)
    sb = lax.broadcasted_iota(jnp.int32, (NB, 1), 0) * BLK
    startsnb = jnp.broadcast_to(starts, (NB, E))
    be_ref[...] = jnp.sum((startsnb <= sb).astype(jnp.int32),
                          axis=1, keepdims=True) - 1
    total = jnp.sum(padded, axis=1, keepdims=True)
    bv_ref[...] = (sb < total).astype(jnp.int32)


def _plan(idx):
    return pl.pallas_call(
        _plan_body,
        out_shape=[
            jax.ShapeDtypeStruct((T, K), jnp.int32),
            jax.ShapeDtypeStruct((NB, 1), jnp.int32),
            jax.ShapeDtypeStruct((NB, 1), jnp.int32),
        ],
    )(idx)


DCH = 32
DNC = RPW // DCH


def _dispatch_body(pos_hbm, xi_hbm, xs_hbm, pos_v, ids_v, idx0, idx1,
                   buf0, buf1, sem_g, sem_s):
    wid = lax.axis_index("s") * NC + lax.axis_index("c")
    pltpu.sync_copy(pos_hbm, pos_v)
    lanes = lax.iota(jnp.int32, L)

    def scatter_i(i, carry):
        pchunk = pos_v[pl.ds(i * L, L)]
        toks = (i * L + lanes) >> 1
        plsc.store_scatter(ids_v, [pchunk], toks)
        return carry

    lax.fori_loop(0, (T * K) // L, scatter_i, 0)

    base = wid * RPW
    idx_bufs = (idx0, idx1)
    bufs = (buf0, buf1)

    def build_ids(j):
        ib = idx_bufs[j % 2]
        for q in range(DCH // L):
            ids16 = ids_v[pl.ds(base + j * DCH + q * L, L)]
            ib[pl.ds(q * L, L)] = jnp.clip(ids16, 0, T - 1)
        return ib

    build_ids(0)
    g_prev = pltpu.async_copy(xi_hbm.at[idx0], buf0, sem_g)
    s_prev = None
    for j in range(DNC):
        if j + 1 < DNC:
            if s_prev is not None:
                s_prev.wait()
            ib = build_ids(j + 1)
            g_next = pltpu.async_copy(xi_hbm.at[ib], bufs[(j + 1) % 2], sem_g)
        g_prev.wait()
        s_cur = pltpu.async_copy(bufs[j % 2],
                                 xs_hbm.at[pl.ds(base + j * DCH, DCH)], sem_s)
        if j + 1 < DNC:
            s_prev, g_prev = s_cur, g_next
        else:
            s_cur.wait()
            if s_prev is not None:
                s_prev.wait()


def _dispatch(pos_flat, x_i32):
    mesh = plsc.VectorSubcoreMesh(core_axis_name="c", subcore_axis_name="s")
    return pl.kernel(
        _dispatch_body,
        out_type=jax.ShapeDtypeStruct((P, D // 2), jnp.int32),
        mesh=mesh,
        compiler_params=pltpu.CompilerParams(needs_layout_passes=False),
        scratch_types=[
            pltpu.VMEM((T * K,), jnp.int32),
            pltpu.VMEM((P,), jnp.int32),
            pltpu.VMEM((DCH,), jnp.int32),
            pltpu.VMEM((DCH,), jnp.int32),
            pltpu.VMEM((DCH, D // 2), jnp.int32),
            pltpu.VMEM((DCH, D // 2), jnp.int32),
            pltpu.SemaphoreType.DMA,
            pltpu.SemaphoreType.DMA,
        ],
    )(pos_flat, x_i32)


def _ffn_body(be_ref, bv_ref, xs_ref, wg_ref, wu_ref, wd_ref, h_ref):
    f = pl.program_id(1)
    b = pl.program_id(0)

    @pl.when(f == 0)
    def _():
        h_ref[...] = jnp.zeros_like(h_ref)

    @pl.when(bv_ref[b] != 0)
    def _():
        xb = xs_ref[...]
        wg = wg_ref[0].astype(jnp.bfloat16)
        wu = wu_ref[0].astype(jnp.bfloat16)
        wd = wd_ref[0].astype(jnp.bfloat16)
        g = lax.dot_general(xb, wg, (((1,), (1,)), ((), ())),
                            preferred_element_type=jnp.float32)
        u = lax.dot_general(xb, wu, (((1,), (1,)), ((), ())),
                            preferred_element_type=jnp.float32)
        a = (g * jax.nn.sigmoid(g) * u).astype(jnp.bfloat16)
        h_ref[...] += lax.dot_general(a, wd, (((1,), (1,)), ((), ())),
                                      preferred_element_type=jnp.float32)


def _ffn(be, bv, xs, wg, wu, wd):
    grid_spec = pltpu.PrefetchScalarGridSpec(
        num_scalar_prefetch=2,
        grid=(NB, NF),
        in_specs=[
            pl.BlockSpec((BLK, D), lambda b, f, be, bv: (b, 0)),
            pl.BlockSpec((1, FT, D), lambda b, f, be, bv: (be[b], f, 0)),
            pl.BlockSpec((1, FT, D), lambda b, f, be, bv: (be[b], f, 0)),
            pl.BlockSpec((1, D, FT), lambda b, f, be, bv: (be[b], 0, f)),
        ],
        out_specs=pl.BlockSpec((BLK, D), lambda b, f, be, bv: (b, 0)),
    )
    return pl.pallas_call(
        _ffn_body,
        grid_spec=grid_spec,
        out_shape=jax.ShapeDtypeStruct((P, D), jnp.float32),
        compiler_params=pltpu.CompilerParams(
            dimension_semantics=("arbitrary", "arbitrary")),
    )(be, bv, xs, wg, wu, wd)


CPW = (T * K) // NW
CCH = 16
CNC = CPW // CCH


def _hexp_body(pos_hbm, h_hbm, hexp_hbm, pos_v, buf0, buf1, sem_g, sem_s):
    wid = lax.axis_index("s") * NC + lax.axis_index("c")
    base = wid * CPW
    pltpu.sync_copy(pos_hbm.at[pl.ds(base, CPW)], pos_v)
    bufs = (buf0, buf1)

    g_prev = pltpu.async_copy(h_hbm.at[pos_v[pl.ds(0, CCH)]], buf0, sem_g)
    s_prev = None
    for j in range(CNC):
        if j + 1 < CNC:
            if s_prev is not None:
                s_prev.wait()
            g_next = pltpu.async_copy(
                h_hbm.at[pos_v[pl.ds((j + 1) * CCH, CCH)]],
                bufs[(j + 1) % 2], sem_g)
        g_prev.wait()
        s_cur = pltpu.async_copy(bufs[j % 2],
                                 hexp_hbm.at[pl.ds(base + j * CCH, CCH)],
                                 sem_s)
        if j + 1 < CNC:
            s_prev, g_prev = s_cur, g_next
        else:
            s_cur.wait()
            if s_prev is not None:
                s_prev.wait()


def _hexp(pos_flat, h):
    mesh = plsc.VectorSubcoreMesh(core_axis_name="c", subcore_axis_name="s")
    return pl.kernel(
        _hexp_body,
        out_type=jax.ShapeDtypeStruct((T * K, D), jnp.float32),
        mesh=mesh,
        compiler_params=pltpu.CompilerParams(needs_layout_passes=False),
        scratch_types=[
            pltpu.VMEM((CPW,), jnp.int32),
            pltpu.VMEM((CCH, D), jnp.float32),
            pltpu.VMEM((CCH, D), jnp.float32),
            pltpu.SemaphoreType.DMA,
            pltpu.SemaphoreType.DMA,
        ],
    )(pos_flat, h)


def _wsum_body(w_ref, he_ref, out_ref):
    he = he_ref[...]
    out_ref[...] = (w_ref[:, 0:1] * he[:, 0, :] + w_ref[:, 1:2] * he[:, 1, :])


def _wsum(wt, hexp):
    RB = 512
    return pl.pallas_call(
        _wsum_body,
        grid=(T // RB,),
        in_specs=[
            pl.BlockSpec((RB, K), lambda r: (r, 0)),
            pl.BlockSpec((RB, K, D), lambda r: (r, 0, 0)),
        ],
        out_specs=pl.BlockSpec((RB, D), lambda r: (r, 0)),
        out_shape=jax.ShapeDtypeStruct((T, D), jnp.float32),
    )(wt, hexp)


@jax.jit
def kernel(hidden_states, W_router, Wg, Wu, Wd):
    bsz, seq, _ = hidden_states.shape
    x = hidden_states.reshape(T, D)
    logits, wt, tidx = _router(x, W_router)
    pos, be, bv = _plan(tidx)
    x_i32 = lax.bitcast_convert_type(
        x.astype(jnp.bfloat16).reshape(T, D // 2, 2), jnp.int32)
    xs_i32 = _dispatch(pos.reshape(-1), x_i32)
    xs = lax.bitcast_convert_type(xs_i32, jnp.bfloat16).reshape(P, D)
    h = _ffn(be.reshape(NB), bv.reshape(NB), xs, Wg, Wu, Wd)
    hexp = _hexp(pos.reshape(-1), h)
    out = _wsum(wt, hexp.reshape(T, K, D))
    return out.reshape(bsz, seq, D), logits


# speedup vs baseline: 1.8666x; 1.4817x over previous
import functools

import jax
import jax.numpy as jnp
from jax import lax
from jax.experimental import pallas as pl
from jax.experimental.pallas import tpu as pltpu
from jax.experimental.pallas import tpu_sc as plsc

D = 2048
F = 4096
E = 8
K = 2
T = 4096
BLK = 512
NB = 24
P = NB * BLK
FT = 512
NF = F // FT

NC = 2
NS = 16
NW = NC * NS
L = 16

RPW = P // NW
TPW = T // NW


def _router_body(x_ref, wr_ref, logits_ref, wt_ref, idx_ref):
    xb = x_ref[...]
    wr = wr_ref[...]
    logits = lax.dot_general(
        xb.astype(jnp.bfloat16), wr.astype(jnp.bfloat16),
        (((1,), (1,)), ((), ())),
        preferred_element_type=jnp.float32,
    )
    m = jnp.max(logits, axis=1, keepdims=True)
    p = jnp.exp(logits - m)
    probs = p / jnp.sum(p, axis=1, keepdims=True)
    eio = lax.broadcasted_iota(jnp.int32, probs.shape, 1)
    w1 = jnp.max(probs, axis=1, keepdims=True)
    i1 = jnp.min(jnp.where(probs == w1, eio, E), axis=1, keepdims=True)
    probs2 = jnp.where(eio == i1, -1.0, probs)
    w2 = jnp.max(probs2, axis=1, keepdims=True)
    i2 = jnp.min(jnp.where(probs2 == w2, eio, E), axis=1, keepdims=True)
    logits_ref[...] = logits
    wt_ref[...] = jnp.concatenate([w1, w2], axis=1)
    idx_ref[...] = jnp.concatenate([i1, i2], axis=1)


def _router(x, wr):
    RB = 1024
    return pl.pallas_call(
        _router_body,
        grid=(T // RB,),
        in_specs=[
            pl.BlockSpec((RB, D), lambda r: (r, 0)),
            pl.BlockSpec((E, D), lambda r: (0, 0)),
        ],
        out_specs=[
            pl.BlockSpec((RB, E), lambda r: (r, 0)),
            pl.BlockSpec((RB, K), lambda r: (r, 0)),
            pl.BlockSpec((RB, K), lambda r: (r, 0)),
        ],
        out_shape=[
            jax.ShapeDtypeStruct((T, E), jnp.float32),
            jax.ShapeDtypeStruct((T, K), jnp.float32),
            jax.ShapeDtypeStruct((T, K), jnp.int32),
        ],
    )(x, wr)


def _plan_body(idx_ref, pos_ref, be_ref, bv_ref):
    i1 = idx_ref[:, 0:1]
    i2 = idx_ref[:, 1:2]
    eio = lax.broadcasted_iota(jnp.int32, (T, E), 1)
    m = ((i1 == eio) | (i2 == eio)).astype(jnp.int32)
    cums = m
    sh = 1
    while sh < T:
        shifted = jnp.concatenate(
            [jnp.zeros((sh, E), jnp.int32), cums[: T - sh, :]], axis=0)
        cums = cums + shifted
        sh *= 2
    counts = cums[T - 1:T, :]
    padded = ((counts + BLK - 1) // BLK) * BLK
    tri = (lax.broadcasted_iota(jnp.int32, (E, E), 0)
           < lax.broadcasted_iota(jnp.int32, (E, E), 1)).astype(jnp.float32)
    starts = lax.dot_general(
        padded.astype(jnp.float32), tri, (((1,), (0,)), ((), ())),
        preferred_element_type=jnp.float32).astype(jnp.int32)
    startsb = jnp.broadcast_to(starts, (T, E))
    c1 = jnp.sum(jnp.where(eio == i1, cums, 0), axis=1, keepdims=True)
    s1 = jnp.sum(jnp.where(eio == i1, startsb, 0), axis=1, keepdims=True)
    c2 = jnp.sum(jnp.where(eio == i2, cums, 0), axis=1, keepdims=True)
    s2 = jnp.sum(jnp.where(eio == i2, startsb, 0), axis=1, keepdims=True)
    pos_ref[...] = jnp.concatenate([s1 + c1 - 1, s2 + c2 - 1], axis=1)
    sb = lax.broadcasted_iota(jnp.int32, (NB, 1), 0) * BLK
    startsnb = jnp.broadcast_to(starts, (NB, E))
    be_ref[...] = jnp.sum((startsnb <= sb).astype(jnp.int32),
                          axis=1, keepdims=True) - 1
    total = jnp.sum(padded, axis=1, keepdims=True)
    bv_ref[...] = (sb < total).astype(jnp.int32)


def _plan(idx):
    return pl.pallas_call(
        _plan_body,
        out_shape=[
            jax.ShapeDtypeStruct((T, K), jnp.int32),
            jax.ShapeDtypeStruct((NB, 1), jnp.int32),
            jax.ShapeDtypeStruct((NB, 1), jnp.int32),
        ],
    )(idx)


DCH = 16
DNC = RPW // DCH


def _dispatch_body(pos_hbm, xi_hbm, xs_hbm, pos_v, ids_v, idx0, idx1,
                   buf0, buf1, sem_g, sem_s):
    wid = lax.axis_index("s") * NC + lax.axis_index("c")
    pltpu.sync_copy(pos_hbm, pos_v)
    lanes = lax.iota(jnp.int32, L)

    UNROLL = 8

    def scatter_i(i, carry):
        for q in range(UNROLL):
            pchunk = pos_v[pl.ds((i * UNROLL + q) * L, L)]
            toks = ((i * UNROLL + q) * L + lanes) >> 1
            plsc.store_scatter(ids_v, [pchunk], toks)
        return carry

    lax.fori_loop(0, (T * K) // (L * UNROLL), scatter_i, 0)

    base = wid * RPW
    idx_bufs = (idx0, idx1)
    bufs = (buf0, buf1)

    def build_ids(j):
        ib = idx_bufs[j % 2]
        for q in range(DCH // L):
            ids16 = ids_v[pl.ds(base + j * DCH + q * L, L)]
            ib[pl.ds(q * L, L)] = jnp.clip(ids16, 0, T - 1)
        return ib

    build_ids(0)
    g_prev = pltpu.async_copy(xi_hbm.at[idx0], buf0, sem_g)
    s_prev = None
    for j in range(DNC):
        if j + 1 < DNC:
            if s_prev is not None:
                s_prev.wait()
            ib = build_ids(j + 1)
            g_next = pltpu.async_copy(xi_hbm.at[ib], bufs[(j + 1) % 2], sem_g)
        g_prev.wait()
        s_cur = pltpu.async_copy(bufs[j % 2],
                                 xs_hbm.at[pl.ds(base + j * DCH, DCH)], sem_s)
        if j + 1 < DNC:
            s_prev, g_prev = s_cur, g_next
        else:
            s_cur.wait()
            if s_prev is not None:
                s_prev.wait()


def _dispatch(pos_flat, x):
    mesh = plsc.VectorSubcoreMesh(core_axis_name="c", subcore_axis_name="s")
    return pl.kernel(
        _dispatch_body,
        out_type=jax.ShapeDtypeStruct((P, D), jnp.float32),
        mesh=mesh,
        compiler_params=pltpu.CompilerParams(needs_layout_passes=False),
        scratch_types=[
            pltpu.VMEM((T * K,), jnp.int32),
            pltpu.VMEM((P,), jnp.int32),
            pltpu.VMEM((DCH,), jnp.int32),
            pltpu.VMEM((DCH,), jnp.int32),
            pltpu.VMEM((DCH, D), jnp.float32),
            pltpu.VMEM((DCH, D), jnp.float32),
            pltpu.SemaphoreType.DMA,
            pltpu.SemaphoreType.DMA,
        ],
    )(pos_flat, x)


def _ffn_body(be_ref, bv_ref, xs_ref, wg_ref, wu_ref, wd_ref, h_ref):
    f = pl.program_id(1)
    b = pl.program_id(0)

    @pl.when(f == 0)
    def _():
        h_ref[...] = jnp.zeros_like(h_ref)

    @pl.when(bv_ref[b] != 0)
    def _():
        xb = xs_ref[...].astype(jnp.bfloat16)
        wg = wg_ref[0].astype(jnp.bfloat16)
        wu = wu_ref[0].astype(jnp.bfloat16)
        wd = wd_ref[0].astype(jnp.bfloat16)
        g = lax.dot_general(xb, wg, (((1,), (1,)), ((), ())),
                            preferred_element_type=jnp.float32)
        u = lax.dot_general(xb, wu, (((1,), (1,)), ((), ())),
                            preferred_element_type=jnp.float32)
        a = (g * jax.nn.sigmoid(g) * u).astype(jnp.bfloat16)
        h_ref[...] += lax.dot_general(a, wd, (((1,), (1,)), ((), ())),
                                      preferred_element_type=jnp.float32)


def _ffn(be, bv, xs, wg, wu, wd):
    grid_spec = pltpu.PrefetchScalarGridSpec(
        num_scalar_prefetch=2,
        grid=(NB, NF),
        in_specs=[
            pl.BlockSpec((BLK, D), lambda b, f, be, bv: (b, 0)),
            pl.BlockSpec((1, FT, D), lambda b, f, be, bv: (be[b], f, 0)),
            pl.BlockSpec((1, FT, D), lambda b, f, be, bv: (be[b], f, 0)),
            pl.BlockSpec((1, D, FT), lambda b, f, be, bv: (be[b], 0, f)),
        ],
        out_specs=pl.BlockSpec((BLK, D), lambda b, f, be, bv: (b, 0)),
    )
    return pl.pallas_call(
        _ffn_body,
        grid_spec=grid_spec,
        out_shape=jax.ShapeDtypeStruct((P, D), jnp.float32),
        compiler_params=pltpu.CompilerParams(
            dimension_semantics=("arbitrary", "arbitrary")),
    )(be, bv, xs, wg, wu, wd)


CPW = (T * K) // NW
CCH = 16
CNC = CPW // CCH


def _hexp_body(pos_hbm, h_hbm, hexp_hbm, pos_v, buf0, buf1, sem_g, sem_s):
    wid = lax.axis_index("s") * NC + lax.axis_index("c")
    base = wid * CPW
    pltpu.sync_copy(pos_hbm.at[pl.ds(base, CPW)], pos_v)
    bufs = (buf0, buf1)

    g_prev = pltpu.async_copy(h_hbm.at[pos_v[pl.ds(0, CCH)]], buf0, sem_g)
    s_prev = None
    for j in range(CNC):
        if j + 1 < CNC:
            if s_prev is not None:
                s_prev.wait()
            g_next = pltpu.async_copy(
                h_hbm.at[pos_v[pl.ds((j + 1) * CCH, CCH)]],
                bufs[(j + 1) % 2], sem_g)
        g_prev.wait()
        s_cur = pltpu.async_copy(bufs[j % 2],
                                 hexp_hbm.at[pl.ds(base + j * CCH, CCH)],
                                 sem_s)
        if j + 1 < CNC:
            s_prev, g_prev = s_cur, g_next
        else:
            s_cur.wait()
            if s_prev is not None:
                s_prev.wait()


def _hexp(pos_flat, h):
    mesh = plsc.VectorSubcoreMesh(core_axis_name="c", subcore_axis_name="s")
    return pl.kernel(
        _hexp_body,
        out_type=jax.ShapeDtypeStruct((T * K, D), jnp.float32),
        mesh=mesh,
        compiler_params=pltpu.CompilerParams(needs_layout_passes=False),
        scratch_types=[
            pltpu.VMEM((CPW,), jnp.int32),
            pltpu.VMEM((CCH, D), jnp.float32),
            pltpu.VMEM((CCH, D), jnp.float32),
            pltpu.SemaphoreType.DMA,
            pltpu.SemaphoreType.DMA,
        ],
    )(pos_flat, h)


def _wsum_body(w_ref, he_ref, out_ref):
    he = he_ref[...]
    out_ref[...] = (w_ref[:, 0:1] * he[:, 0, :] + w_ref[:, 1:2] * he[:, 1, :])


def _wsum(wt, hexp):
    RB = 512
    return pl.pallas_call(
        _wsum_body,
        grid=(T // RB,),
        in_specs=[
            pl.BlockSpec((RB, K), lambda r: (r, 0)),
            pl.BlockSpec((RB, K, D), lambda r: (r, 0, 0)),
        ],
        out_specs=pl.BlockSpec((RB, D), lambda r: (r, 0)),
        out_shape=jax.ShapeDtypeStruct((T, D), jnp.float32),
    )(wt, hexp)


@jax.jit
def kernel(hidden_states, W_router, Wg, Wu, Wd):
    bsz, seq, _ = hidden_states.shape
    x = hidden_states.reshape(T, D)
    logits, wt, tidx = _router(x, W_router)
    pos, be, bv = _plan(tidx)
    xs = _dispatch(pos.reshape(-1), x)
    h = _ffn(be.reshape(NB), bv.reshape(NB), xs, Wg, Wu, Wd)
    hexp = _hexp(pos.reshape(-1), h)
    out = _wsum(wt, hexp.reshape(T, K, D))
    return out.reshape(bsz, seq, D), logits


# speedup vs baseline: 1.8726x; 1.0032x over previous
import functools

import jax
import jax.numpy as jnp
from jax import lax
from jax.experimental import pallas as pl
from jax.experimental.pallas import tpu as pltpu
from jax.experimental.pallas import tpu_sc as plsc

D = 2048
F = 4096
E = 8
K = 2
T = 4096
BLK = 512
NB = 24
P = NB * BLK
FT = 512
NF = F // FT

NC = 2
NS = 16
NW = NC * NS
L = 16

RPW = P // NW
TPW = T // NW


def _router_body(x_ref, wr_ref, logits_ref, wt_ref, idx_ref):
    xb = x_ref[...]
    wr = wr_ref[...]
    logits = lax.dot_general(
        xb.astype(jnp.bfloat16), wr.astype(jnp.bfloat16),
        (((1,), (1,)), ((), ())),
        preferred_element_type=jnp.float32,
    )
    m = jnp.max(logits, axis=1, keepdims=True)
    p = jnp.exp(logits - m)
    probs = p / jnp.sum(p, axis=1, keepdims=True)
    eio = lax.broadcasted_iota(jnp.int32, probs.shape, 1)
    w1 = jnp.max(probs, axis=1, keepdims=True)
    i1 = jnp.min(jnp.where(probs == w1, eio, E), axis=1, keepdims=True)
    probs2 = jnp.where(eio == i1, -1.0, probs)
    w2 = jnp.max(probs2, axis=1, keepdims=True)
    i2 = jnp.min(jnp.where(probs2 == w2, eio, E), axis=1, keepdims=True)
    logits_ref[...] = logits
    wt_ref[...] = jnp.concatenate([w1, w2], axis=1)
    idx_ref[...] = jnp.concatenate([i1, i2], axis=1)


def _router(x, wr):
    RB = 1024
    return pl.pallas_call(
        _router_body,
        grid=(T // RB,),
        in_specs=[
            pl.BlockSpec((RB, D), lambda r: (r, 0)),
            pl.BlockSpec((E, D), lambda r: (0, 0)),
        ],
        out_specs=[
            pl.BlockSpec((RB, E), lambda r: (r, 0)),
            pl.BlockSpec((RB, K), lambda r: (r, 0)),
            pl.BlockSpec((RB, K), lambda r: (r, 0)),
        ],
        out_shape=[
            jax.ShapeDtypeStruct((T, E), jnp.float32),
            jax.ShapeDtypeStruct((T, K), jnp.float32),
            jax.ShapeDtypeStruct((T, K), jnp.int32),
        ],
    )(x, wr)


def _plan_body(idx_ref, pos_ref, be_ref, bv_ref):
    i1 = idx_ref[:, 0:1]
    i2 = idx_ref[:, 1:2]
    eio = lax.broadcasted_iota(jnp.int32, (T, E), 1)
    m = ((i1 == eio) | (i2 == eio)).astype(jnp.int32)
    cums = m
    sh = 1
    while sh < T:
        shifted = jnp.concatenate(
            [jnp.zeros((sh, E), jnp.int32), cums[: T - sh, :]], axis=0)
        cums = cums + shifted
        sh *= 2
    counts = cums[T - 1:T, :]
    padded = ((counts + BLK - 1) // BLK) * BLK
    tri = (lax.broadcasted_iota(jnp.int32, (E, E), 0)
           < lax.broadcasted_iota(jnp.int32, (E, E), 1)).astype(jnp.float32)
    starts = lax.dot_general(
        padded.astype(jnp.float32), tri, (((1,), (0,)), ((), ())),
        preferred_element_type=jnp.float32).astype(jnp.int32)
    startsb = jnp.broadcast_to(starts, (T, E))
    c1 = jnp.sum(jnp.where(eio == i1, cums, 0), axis=1, keepdims=True)
    s1 = jnp.sum(jnp.where(eio == i1, startsb, 0), axis=1, keepdims=True)
    c2 = jnp.sum(jnp.where(eio == i2, cums, 0), axis=1, keepdims=True)
    s2 = jnp.sum(jnp.where(eio == i2, startsb, 0), axis=1, keepdims=True)
    pos_ref[...] = jnp.concatenate([s1 + c1 - 1, s2 + c2 - 1], axis=1)
    sb = lax.broadcasted_iota(jnp.int32, (NB, 1), 0) * BLK
    startsnb = jnp.broadcast_to(starts, (NB, E))
    be_ref[...] = jnp.sum((startsnb <= sb).astype(jnp.int32),
                          axis=1, keepdims=True) - 1
    total = jnp.sum(padded, axis=1, keepdims=True)
    bv_ref[...] = (sb < total).astype(jnp.int32)


def _plan(idx):
    return pl.pallas_call(
        _plan_body,
        out_shape=[
            jax.ShapeDtypeStruct((T, K), jnp.int32),
            jax.ShapeDtypeStruct((NB, 1), jnp.int32),
            jax.ShapeDtypeStruct((NB, 1), jnp.int32),
        ],
    )(idx)


DCH = 16
DNC = RPW // DCH


PPS = (T * K) // NS


def _dispatch_body(pos_hbm, xi_hbm, xs_hbm, pos_v, toks_v, ids_v, shared_ids,
                   idx0, idx1, buf0, buf1, sem_g, sem_s):
    sid = lax.axis_index("s")
    wid = sid * NC + lax.axis_index("c")
    pltpu.sync_copy(pos_hbm.at[pl.ds(sid * PPS, PPS)], pos_v)
    lanes = lax.iota(jnp.int32, L)
    for q in range(PPS // L):
        toks_v[pl.ds(q * L, L)] = (sid * PPS + q * L + lanes) >> 1
    pltpu.async_copy(toks_v, shared_ids.at[pos_v], sem_s).wait()
    plsc.subcore_barrier()
    base = wid * RPW
    pltpu.sync_copy(shared_ids.at[pl.ds(base, RPW)], ids_v)

    idx_bufs = (idx0, idx1)
    bufs = (buf0, buf1)

    def build_ids(j):
        ib = idx_bufs[j % 2]
        for q in range(DCH // L):
            ids16 = ids_v[pl.ds(j * DCH + q * L, L)]
            ib[pl.ds(q * L, L)] = jnp.clip(ids16, 0, T - 1)
        return ib

    build_ids(0)
    g_prev = pltpu.async_copy(xi_hbm.at[idx0], buf0, sem_g)
    s_prev = None
    for j in range(DNC):
        if j + 1 < DNC:
            if s_prev is not None:
                s_prev.wait()
            ib = build_ids(j + 1)
            g_next = pltpu.async_copy(xi_hbm.at[ib], bufs[(j + 1) % 2], sem_g)
        g_prev.wait()
        s_cur = pltpu.async_copy(bufs[j % 2],
                                 xs_hbm.at[pl.ds(base + j * DCH, DCH)], sem_s)
        if j + 1 < DNC:
            s_prev, g_prev = s_cur, g_next
        else:
            s_cur.wait()
            if s_prev is not None:
                s_prev.wait()


def _dispatch(pos_flat, x):
    mesh = plsc.VectorSubcoreMesh(core_axis_name="c", subcore_axis_name="s")
    return pl.kernel(
        _dispatch_body,
        out_type=jax.ShapeDtypeStruct((P, D), jnp.float32),
        mesh=mesh,
        compiler_params=pltpu.CompilerParams(needs_layout_passes=False),
        scratch_types=[
            pltpu.VMEM((PPS,), jnp.int32),
            pltpu.VMEM((PPS,), jnp.int32),
            pltpu.VMEM((RPW,), jnp.int32),
            pltpu.VMEM_SHARED((P,), jnp.int32),
            pltpu.VMEM((DCH,), jnp.int32),
            pltpu.VMEM((DCH,), jnp.int32),
            pltpu.VMEM((DCH, D), jnp.float32),
            pltpu.VMEM((DCH, D), jnp.float32),
            pltpu.SemaphoreType.DMA,
            pltpu.SemaphoreType.DMA,
        ],
    )(pos_flat, x)


def _ffn_body(be_ref, bv_ref, xs_ref, wg_ref, wu_ref, wd_ref, h_ref):
    f = pl.program_id(1)
    b = pl.program_id(0)

    @pl.when(f == 0)
    def _():
        h_ref[...] = jnp.zeros_like(h_ref)

    @pl.when(bv_ref[b] != 0)
    def _():
        xb = xs_ref[...].astype(jnp.bfloat16)
        wg = wg_ref[0].astype(jnp.bfloat16)
        wu = wu_ref[0].astype(jnp.bfloat16)
        wd = wd_ref[0].astype(jnp.bfloat16)
        g = lax.dot_general(xb, wg, (((1,), (1,)), ((), ())),
                            preferred_element_type=jnp.float32)
        u = lax.dot_general(xb, wu, (((1,), (1,)), ((), ())),
                            preferred_element_type=jnp.float32)
        a = (g * jax.nn.sigmoid(g) * u).astype(jnp.bfloat16)
        h_ref[...] += lax.dot_general(a, wd, (((1,), (1,)), ((), ())),
                                      preferred_element_type=jnp.float32)


def _ffn(be, bv, xs, wg, wu, wd):
    grid_spec = pltpu.PrefetchScalarGridSpec(
        num_scalar_prefetch=2,
        grid=(NB, NF),
        in_specs=[
            pl.BlockSpec((BLK, D), lambda b, f, be, bv: (b, 0)),
            pl.BlockSpec((1, FT, D), lambda b, f, be, bv: (be[b], f, 0)),
            pl.BlockSpec((1, FT, D), lambda b, f, be, bv: (be[b], f, 0)),
            pl.BlockSpec((1, D, FT), lambda b, f, be, bv: (be[b], 0, f)),
        ],
        out_specs=pl.BlockSpec((BLK, D), lambda b, f, be, bv: (b, 0)),
    )
    return pl.pallas_call(
        _ffn_body,
        grid_spec=grid_spec,
        out_shape=jax.ShapeDtypeStruct((P, D), jnp.float32),
        compiler_params=pltpu.CompilerParams(
            dimension_semantics=("arbitrary", "arbitrary")),
    )(be, bv, xs, wg, wu, wd)


CPW = (T * K) // NW
CCH = 16
CNC = CPW // CCH


def _hexp_body(pos_hbm, h_hbm, hexp_hbm, pos_v, buf0, buf1, sem_g, sem_s):
    wid = lax.axis_index("s") * NC + lax.axis_index("c")
    base = wid * CPW
    pltpu.sync_copy(pos_hbm.at[pl.ds(base, CPW)], pos_v)
    bufs = (buf0, buf1)

    g_prev = pltpu.async_copy(h_hbm.at[pos_v[pl.ds(0, CCH)]], buf0, sem_g)
    s_prev = None
    for j in range(CNC):
        if j + 1 < CNC:
            if s_prev is not None:
                s_prev.wait()
            g_next = pltpu.async_copy(
                h_hbm.at[pos_v[pl.ds((j + 1) * CCH, CCH)]],
                bufs[(j + 1) % 2], sem_g)
        g_prev.wait()
        s_cur = pltpu.async_copy(bufs[j % 2],
                                 hexp_hbm.at[pl.ds(base + j * CCH, CCH)],
                                 sem_s)
        if j + 1 < CNC:
            s_prev, g_prev = s_cur, g_next
        else:
            s_cur.wait()
            if s_prev is not None:
                s_prev.wait()


def _hexp(pos_flat, h):
    mesh = plsc.VectorSubcoreMesh(core_axis_name="c", subcore_axis_name="s")
    return pl.kernel(
        _hexp_body,
        out_type=jax.ShapeDtypeStruct((T * K, D), jnp.float32),
        mesh=mesh,
        compiler_params=pltpu.CompilerParams(needs_layout_passes=False),
        scratch_types=[
            pltpu.VMEM((CPW,), jnp.int32),
            pltpu.VMEM((CCH, D), jnp.float32),
            pltpu.VMEM((CCH, D), jnp.float32),
            pltpu.SemaphoreType.DMA,
            pltpu.SemaphoreType.DMA,
        ],
    )(pos_flat, h)


def _wsum_body(w_ref, he_ref, out_ref):
    he = he_ref[...]
    out_ref[...] = (w_ref[:, 0:1] * he[:, 0, :] + w_ref[:, 1:2] * he[:, 1, :])


def _wsum(wt, hexp):
    RB = 512
    return pl.pallas_call(
        _wsum_body,
        grid=(T // RB,),
        in_specs=[
            pl.BlockSpec((RB, K), lambda r: (r, 0)),
            pl.BlockSpec((RB, K, D), lambda r: (r, 0, 0)),
        ],
        out_specs=pl.BlockSpec((RB, D), lambda r: (r, 0)),
        out_shape=jax.ShapeDtypeStruct((T, D), jnp.float32),
    )(wt, hexp)


@jax.jit
def kernel(hidden_states, W_router, Wg, Wu, Wd):
    bsz, seq, _ = hidden_states.shape
    x = hidden_states.reshape(T, D)
    logits, wt, tidx = _router(x, W_router)
    pos, be, bv = _plan(tidx)
    xs = _dispatch(pos.reshape(-1), x)
    h = _ffn(be.reshape(NB), bv.reshape(NB), xs, Wg, Wu, Wd)
    hexp = _hexp(pos.reshape(-1), h)
    out = _wsum(wt, hexp.reshape(T, K, D))
    return out.reshape(bsz, seq, D), logits
